# baseline (speedup 1.0000x reference)
import functools

import jax
import jax.numpy as jnp
import numpy as np
from jax.experimental import pallas as pl
from jax.experimental.pallas import tpu as pltpu

BQ = 256
WS = 1536
WS2 = 768


def _fp_block(sref, pos_t_ref, batch_ref, pos_skip_ref, batch_skip_ref,
              x_ref, x_skip_ref, W1_ref, b1_ref, W2_ref, b2_ref, out_ref,
              *, n1, d, k):
    f32 = jnp.float32
    bf16 = jnp.bfloat16
    i = pl.program_id(0)
    base = pl.multiple_of(sref[0, i], 128)
    fast = sref[1, i]

    q = pos_skip_ref[...]
    qx = q[:, 0:1]
    qy = q[:, 1:2]
    qz = q[:, 2:3]
    qq = qx * qx + qy * qy + qz * qz
    qb = q.astype(bf16) * bf16(-2.0)
    bq = batch_skip_ref[...]

    def scan(col0, ws):
        pos3 = pos_t_ref[:, pl.ds(col0, ws)]
        px = pos3[0:1, :]
        py = pos3[1:2, :]
        pz = pos3[2:3, :]
        pp = px * px + py * py + pz * pz
        pb = pos3.astype(bf16)
        qp2 = jax.lax.dot_general(qb, pb, (((1,), (0,)), ((), ())),
                                  preferred_element_type=f32)
        bp = batch_ref[0:1, pl.ds(col0, ws)]
        bids = jax.lax.broadcasted_iota(jnp.int32, (8, 1), 0)
        ohq = (bq == bids.reshape(1, 8)).astype(bf16)
        ohp = (bp == bids).astype(bf16)
        same = jax.lax.dot_general(ohq, ohp, (((1,), (0,)), ((), ())),
                                   preferred_element_type=f32)
        d2 = jnp.maximum(qq + pp + qp2, 0.0) + (1.0 - same) * f32(1e10)

        colids = jax.lax.broadcasted_iota(jnp.int32, (BQ, ws), 1) + col0
        bigi = jnp.int32(np.iinfo(np.int32).max)
        wmat = jnp.zeros((BQ, ws), f32)
        wsum = jnp.zeros((BQ, 1), f32)
        for it in range(k):
            m = jnp.min(d2, axis=1, keepdims=True)
            amin = jnp.min(jnp.where(d2 == m, colids, bigi), axis=1,
                           keepdims=True)
            onehot = colids == amin
            w = 1.0 / jnp.maximum(m, f32(1e-16))
            wmat = jnp.where(onehot, w, wmat)
            wsum = wsum + w
            if it < k - 1:
                d2 = jnp.where(onehot, f32(np.inf), d2)

        y = jax.lax.dot_general(wmat, x_ref[pl.ds(col0, ws), :],
                                (((1,), (0,)), ((), ())),
                                preferred_element_type=f32)
        y = y / wsum

        W1a = W1_ref[0:d, :]
        W1b = W1_ref[d:, :]
        h = (jax.lax.dot_general(y, W1a, (((1,), (0,)), ((), ())),
                                 preferred_element_type=f32)
             + jax.lax.dot_general(x_skip_ref[...], W1b,
                                   (((1,), (0,)), ((), ())),
                                   preferred_element_type=f32)
             + b1_ref[...])
        h = jnp.maximum(h, 0.0)
        h = jax.lax.dot_general(h, W2_ref[...], (((1,), (0,)), ((), ())),
                                preferred_element_type=f32) + b2_ref[...]
        out_ref[...] = jnp.maximum(h, 0.0)

    @pl.when(fast == 2)
    def _():
        scan(base, WS2)

    @pl.when(fast == 1)
    def _():
        scan(base, WS)

    @pl.when(fast == 0)
    def _():
        scan(0, n1)


def kernel(x, pos, batch, x_skip, pos_skip, batch_skip, W1, b1, W2, b2):
    n1, d = x.shape
    n2, ds = x_skip.shape
    h_dim = W1.shape[1]
    k = 3
    nb = 8

    pos_t = pos.T
    batch_i = batch.astype(jnp.int32)
    batch_skip_i = batch_skip.astype(jnp.int32)
    batch_r = batch_i.reshape(1, n1)
    batch_skip_r = batch_skip_i.reshape(n2, 1)
    b1r = b1.reshape(1, h_dim)
    b2r = b2.reshape(1, h_dim)

    starts = jnp.searchsorted(batch_i, jnp.arange(nb + 1), side="left")
    counts = starts[1:] - starts[:-1]
    qb0 = batch_skip_i[0::BQ]
    qb1 = batch_skip_i[BQ - 1::BQ]
    s = starts[qb0]
    e = starts[qb1 + 1]
    base1 = jnp.minimum((s // 128) * 128, n1 - WS)
    base2 = jnp.minimum((s // 128) * 128, n1 - WS2)
    cover1 = (e - base1) <= WS
    cover2 = (e - base2) <= WS2
    badcum = jnp.concatenate(
        [jnp.zeros((1,), jnp.int32),
         jnp.cumsum((counts < k).astype(jnp.int32))])
    allok = (badcum[qb1 + 1] - badcum[qb0]) == 0
    fast = jnp.where(cover2 & allok, 2,
                     jnp.where(cover1 & allok, 1, 0)).astype(jnp.int32)
    base = jnp.where(fast == 2, base2,
                     jnp.where(fast == 1, base1, 0))
    sarr = jnp.stack([base.astype(jnp.int32), fast])

    grid = (n2 // BQ,)
    full = lambda i, sref: (0, 0)
    h = pl.pallas_call(
        functools.partial(_fp_block, n1=n1, d=d, k=k),
        grid_spec=pltpu.PrefetchScalarGridSpec(
            num_scalar_prefetch=1,
            grid=grid,
            in_specs=[
                pl.BlockSpec((3, n1), full),
                pl.BlockSpec((1, n1), full),
                pl.BlockSpec((BQ, 3), lambda i, s: (i, 0)),
                pl.BlockSpec((BQ, 1), lambda i, s: (i, 0)),
                pl.BlockSpec((n1, d), full),
                pl.BlockSpec((BQ, ds), lambda i, s: (i, 0)),
                pl.BlockSpec((d + ds, h_dim), full),
                pl.BlockSpec((1, h_dim), full),
                pl.BlockSpec((h_dim, h_dim), full),
                pl.BlockSpec((1, h_dim), full),
            ],
            out_specs=pl.BlockSpec((BQ, h_dim), lambda i, s: (i, 0)),
        ),
        out_shape=jax.ShapeDtypeStruct((n2, h_dim), jnp.float32),
    )(sarr, pos_t, batch_r, pos_skip, batch_skip_r, x, x_skip,
      W1, b1r, W2, b2r)
    return (h, pos_skip, batch_skip)


# speedup vs baseline: 1.0629x; 1.0629x over previous
import functools

import jax
import jax.numpy as jnp
import numpy as np
from jax.experimental import pallas as pl
from jax.experimental.pallas import tpu as pltpu

BQ = 256
WS = 1536
WS2 = 768


def _fp_block(sref, pos_t_ref, batch_ref, pos_skip_ref, batch_skip_ref,
              x_ref, x_skip_ref, W1_ref, b1_ref, W2_ref, b2_ref, out_ref,
              *, n1, d, k):
    f32 = jnp.float32
    bf16 = jnp.bfloat16
    i = pl.program_id(0)
    base = pl.multiple_of(sref[0, i], 128)
    fast = sref[1, i]

    q = pos_skip_ref[...]
    qx = q[:, 0:1]
    qy = q[:, 1:2]
    qz = q[:, 2:3]
    qq = qx * qx + qy * qy + qz * qz
    qb = q.astype(bf16) * bf16(-2.0)
    bq = batch_skip_ref[...]

    def scan(col0, ws):
        pos3 = pos_t_ref[:, pl.ds(col0, ws)]
        px = pos3[0:1, :]
        py = pos3[1:2, :]
        pz = pos3[2:3, :]
        pp = px * px + py * py + pz * pz
        pb = pos3.astype(bf16)
        qp2 = jax.lax.dot_general(qb, pb, (((1,), (0,)), ((), ())),
                                  preferred_element_type=f32)
        bp = batch_ref[0:1, pl.ds(col0, ws)]
        d2 = jnp.maximum(qq + pp + qp2, 0.0)
        d2 = d2 + jnp.where(bq != bp, f32(1e10), f32(0.0))

        colids = jax.lax.broadcasted_iota(jnp.int32, (BQ, ws), 1) + col0
        bigi = jnp.int32(np.iinfo(np.int32).max)
        wmat = jnp.zeros((BQ, ws), f32)
        wsum = jnp.zeros((BQ, 1), f32)
        for it in range(k):
            m = jnp.min(d2, axis=1, keepdims=True)
            amin = jnp.min(jnp.where(d2 == m, colids, bigi), axis=1,
                           keepdims=True)
            onehot = colids == amin
            w = 1.0 / jnp.maximum(m, f32(1e-16))
            wmat = jnp.where(onehot, w, wmat)
            wsum = wsum + w
            if it < k - 1:
                d2 = jnp.where(onehot, f32(np.inf), d2)

        y = jax.lax.dot_general(wmat, x_ref[pl.ds(col0, ws), :],
                                (((1,), (0,)), ((), ())),
                                preferred_element_type=f32)
        y = y / wsum

        W1a = W1_ref[0:d, :]
        W1b = W1_ref[d:, :]
        h = (jax.lax.dot_general(y, W1a, (((1,), (0,)), ((), ())),
                                 preferred_element_type=f32)
             + jax.lax.dot_general(x_skip_ref[...], W1b,
                                   (((1,), (0,)), ((), ())),
                                   preferred_element_type=f32)
             + b1_ref[...])
        h = jnp.maximum(h, 0.0)
        h = jax.lax.dot_general(h, W2_ref[...], (((1,), (0,)), ((), ())),
                                preferred_element_type=f32) + b2_ref[...]
        out_ref[...] = jnp.maximum(h, 0.0)

    @pl.when(fast == 2)
    def _():
        scan(base, WS2)

    @pl.when(fast == 1)
    def _():
        scan(base, WS)

    @pl.when(fast == 0)
    def _():
        scan(0, n1)


def kernel(x, pos, batch, x_skip, pos_skip, batch_skip, W1, b1, W2, b2):
    n1, d = x.shape
    n2, ds = x_skip.shape
    h_dim = W1.shape[1]
    k = 3
    nb = 8

    pos_t = pos.T
    batch_i = batch.astype(jnp.int32)
    batch_skip_i = batch_skip.astype(jnp.int32)
    batch_r = batch_i.reshape(1, n1)
    batch_skip_r = batch_skip_i.reshape(n2, 1)
    b1r = b1.reshape(1, h_dim)
    b2r = b2.reshape(1, h_dim)

    starts = jnp.searchsorted(batch_i, jnp.arange(nb + 1), side="left")
    counts = starts[1:] - starts[:-1]
    qb0 = batch_skip_i[0::BQ]
    qb1 = batch_skip_i[BQ - 1::BQ]
    s = starts[qb0]
    e = starts[qb1 + 1]
    base1 = jnp.minimum((s // 128) * 128, n1 - WS)
    base2 = jnp.minimum((s // 128) * 128, n1 - WS2)
    cover1 = (e - base1) <= WS
    cover2 = (e - base2) <= WS2
    badcum = jnp.concatenate(
        [jnp.zeros((1,), jnp.int32),
         jnp.cumsum((counts < k).astype(jnp.int32))])
    allok = (badcum[qb1 + 1] - badcum[qb0]) == 0
    fast = jnp.where(cover2 & allok, 2,
                     jnp.where(cover1 & allok, 1, 0)).astype(jnp.int32)
    base = jnp.where(fast == 2, base2,
                     jnp.where(fast == 1, base1, 0))
    sarr = jnp.stack([base.astype(jnp.int32), fast])

    grid = (n2 // BQ,)
    full = lambda i, sref: (0, 0)
    h = pl.pallas_call(
        functools.partial(_fp_block, n1=n1, d=d, k=k),
        grid_spec=pltpu.PrefetchScalarGridSpec(
            num_scalar_prefetch=1,
            grid=grid,
            in_specs=[
                pl.BlockSpec((3, n1), full),
                pl.BlockSpec((1, n1), full),
                pl.BlockSpec((BQ, 3), lambda i, s: (i, 0)),
                pl.BlockSpec((BQ, 1), lambda i, s: (i, 0)),
                pl.BlockSpec((n1, d), full),
                pl.BlockSpec((BQ, ds), lambda i, s: (i, 0)),
                pl.BlockSpec((d + ds, h_dim), full),
                pl.BlockSpec((1, h_dim), full),
                pl.BlockSpec((h_dim, h_dim), full),
                pl.BlockSpec((1, h_dim), full),
            ],
            out_specs=pl.BlockSpec((BQ, h_dim), lambda i, s: (i, 0)),
        ),
        out_shape=jax.ShapeDtypeStruct((n2, h_dim), jnp.float32),
    )(sarr, pos_t, batch_r, pos_skip, batch_skip_r, x, x_skip,
      W1, b1r, W2, b2r)
    return (h, pos_skip, batch_skip)
